# baseline (speedup 1.0000x reference)
import functools

import jax
import jax.numpy as jnp
from jax import lax
from jax.experimental import pallas as pl
from jax.experimental.pallas import tpu as pltpu
from jax.experimental.pallas import tpu_sc as plsc

_ND = 13
_NF = 26
_EMB = 16
_ROW = _ND + _NF * _EMB

_NC = 2
_NS = 16
_NW = _NC * _NS

_BB = 2048


def _sc_embed(n_batch, seq, f_dim):
    n_half = n_batch // _BB
    n_emb_units = seq * _NF * n_half
    n_dense_units = seq * n_half
    emb_iters = -(-n_emb_units // _NW)
    dense_iters = -(-n_dense_units // _NW)
    mesh = plsc.VectorSubcoreMesh(core_axis_name="c", subcore_axis_name="s")

    @functools.partial(
        pl.kernel,
        mesh=mesh,
        out_type=jax.ShapeDtypeStruct((seq, _ROW, n_batch), jnp.float32),
        scratch_types=[
            pltpu.VMEM((_BB,), jnp.float32),
            pltpu.VMEM((_BB // 128, 128), jnp.int32),
            pltpu.VMEM((_BB, _EMB), jnp.float32),
            pltpu.VMEM((_EMB, _BB), jnp.float32),
            pltpu.VMEM((_ND, _BB), jnp.float32),
            pltpu.SemaphoreType.DMA,
        ],
        compiler_params=pltpu.CompilerParams(
            use_tc_tiling_on_sc=False, needs_layout_passes=False),
    )
    def k(xt, tbl, out, xi_v, idx_v, emb_v, v_out, dense_v, sem):
        wid = lax.axis_index("s") * _NC + lax.axis_index("c")
        lanes = lax.iota(jnp.int32, 16)

        def emb_unit(q):
            s = q // (_NF * n_half)
            r = q % (_NF * n_half)
            f = r // n_half
            b0 = (r % n_half) * _BB
            pltpu.sync_copy(xt.at[s, _ND + f, pl.ds(b0, _BB)], xi_v)
            for j in range(_BB // 128):
                for p in range(8):
                    idx_v[j, pl.ds(p * 16, 16)] = lax.convert_element_type(
                        xi_v[pl.ds(j * 128 + p * 16, 16)], jnp.int32)
            cps = [
                pltpu.async_copy(
                    tbl.at[f].at[idx_v.at[j]],
                    emb_v.at[pl.ds(j * 128, 128)], sem)
                for j in range(_BB // 128)
            ]
            for cp in cps:
                cp.wait()

            def weave(b, carry2):
                bvec = jnp.full((16,), b, dtype=jnp.int32)
                plsc.store_scatter(v_out, [lanes, bvec], emb_v[b])
                return carry2

            lax.fori_loop(0, _BB, weave, 0)
            pltpu.sync_copy(
                v_out, out.at[s, pl.ds(_ND + f * _EMB, _EMB), pl.ds(b0, _BB)])

        def dense_unit(q):
            s = q // n_half
            b0 = (q % n_half) * _BB
            pltpu.sync_copy(xt.at[s, pl.ds(0, _ND), pl.ds(b0, _BB)], dense_v)
            pltpu.sync_copy(
                dense_v, out.at[s, pl.ds(0, _ND), pl.ds(b0, _BB)])

        def emb_loop(k_, carry):
            q = k_ * _NW + wid

            @pl.when(q < n_emb_units)
            def _():
                emb_unit(q)

            return carry

        def dense_loop(k_, carry):
            q = k_ * _NW + wid

            @pl.when(q < n_dense_units)
            def _():
                dense_unit(q)

            return carry

        lax.fori_loop(0, dense_iters, dense_loop, 0)
        lax.fori_loop(0, emb_iters, emb_loop, 0)

    return k


def kernel(x, tables):
    b, seq, f_dim = x.shape
    xt = x.transpose(1, 2, 0)
    out_t = _sc_embed(b, seq, f_dim)(xt, tables)
    return out_t.transpose(2, 0, 1)


# speedup vs baseline: 1.0019x; 1.0019x over previous
import functools

import jax
import jax.numpy as jnp
from jax import lax
from jax.experimental import pallas as pl
from jax.experimental.pallas import tpu as pltpu
from jax.experimental.pallas import tpu_sc as plsc

_ND = 13
_NF = 26
_EMB = 16
_ROW = _ND + _NF * _EMB

_NC = 2
_NS = 16
_NW = _NC * _NS

_BB = 1024
_NG = _BB // 128


def _sc_embed(n_batch, seq, f_dim):
    n_q = n_batch // _BB
    n_emb_units = seq * _NF * n_q
    n_dense_units = seq * n_q
    emb_base, emb_extra = divmod(n_emb_units, _NW)
    dense_base, dense_extra = divmod(n_dense_units, _NW)
    mesh = plsc.VectorSubcoreMesh(core_axis_name="c", subcore_axis_name="s")

    @functools.partial(
        pl.kernel,
        mesh=mesh,
        out_type=jax.ShapeDtypeStruct((seq, _ROW, n_batch), jnp.float32),
        scratch_types=[
            pltpu.VMEM((2, _BB), jnp.float32),
            pltpu.VMEM((2 * _NG, 128), jnp.int32),
            pltpu.VMEM((2 * _BB, _EMB), jnp.float32),
            pltpu.VMEM((2 * _EMB, _BB), jnp.float32),
            pltpu.VMEM((_ND, _BB), jnp.float32),
            pltpu.SemaphoreType.DMA,
            pltpu.SemaphoreType.DMA,
        ],
        compiler_params=pltpu.CompilerParams(
            use_tc_tiling_on_sc=False, needs_layout_passes=False),
    )
    def k(xt, tbl, out, xi_v, idx_v, emb_v, v_out, dense_v, s0, sx):
        wid = lax.axis_index("s") * _NC + lax.axis_index("c")
        lanes = lax.iota(jnp.int32, 16)
        my_units = emb_base + jnp.where(wid < emb_extra, 1, 0)
        my_dense = dense_base + jnp.where(wid < dense_extra, 1, 0)

        def unit_coords(kk):
            q = jnp.minimum(kk, my_units - 1) * _NW + wid
            s = q // (_NF * n_q)
            r = q % (_NF * n_q)
            f = r // n_q
            b0 = (r % n_q) * _BB
            return s, f, b0

        def fetch_xi(kk, slot):
            s, f, b0 = unit_coords(kk)
            return pltpu.async_copy(
                xt.at[s, _ND + f, pl.ds(b0, _BB)],
                xi_v.at[slot], sx)

        def fire(kk, slot, sem):
            s, f, b0 = unit_coords(kk)
            for j in range(_NG):
                for p in range(8):
                    idx_v[slot * _NG + j, pl.ds(p * 16, 16)] = (
                        lax.convert_element_type(
                            xi_v[slot, pl.ds(j * 128 + p * 16, 16)],
                            jnp.int32))
            return [
                pltpu.async_copy(
                    tbl.at[f].at[idx_v.at[slot * _NG + j]],
                    emb_v.at[pl.ds(slot * _BB + j * 128, 128)], sem)
                for j in range(_NG)
            ]

        def weave_write(kk, slot):
            s, f, b0 = unit_coords(kk)

            def weave(bb, carry2):
                base = jnp.full((16,), bb * 16, dtype=jnp.int32)
                for i in range(16):
                    plsc.store_scatter(
                        v_out, [lanes + slot * _EMB, base + i],
                        emb_v[slot * _BB + bb * 16 + i])
                return carry2

            lax.fori_loop(0, _BB // 16, weave, 0)
            pltpu.sync_copy(
                v_out.at[pl.ds(slot * _EMB, _EMB)],
                out.at[s, pl.ds(_ND + f * _EMB, _EMB), pl.ds(b0, _BB)])

        def dense_loop(kk, carry):
            q = jnp.minimum(kk, my_dense - 1) * _NW + wid
            s = q // n_q
            b0 = (q % n_q) * _BB
            pltpu.sync_copy(xt.at[s, pl.ds(0, _ND), pl.ds(b0, _BB)], dense_v)
            pltpu.sync_copy(
                dense_v, out.at[s, pl.ds(0, _ND), pl.ds(b0, _BB)])
            return carry

        lax.fori_loop(0, dense_base + 1, dense_loop, 0)

        iters = emb_base + 1
        fetch_xi(0, 0).wait()
        fire(0, 0, s0)

        def step(kk, carry):
            slot = lax.rem(kk, 2)

            @pl.when(kk + 1 < iters)
            def _():
                fetch_xi(kk + 1, 1 - slot)

            for j in range(_NG):
                pltpu.make_async_copy(
                    tbl.at[0].at[idx_v.at[0]],
                    emb_v.at[pl.ds(0, 128)],
                    s0).wait()
            weave_write(kk, slot)

            @pl.when(kk + 1 < iters)
            def _():
                pltpu.make_async_copy(
                    xt.at[0, 0, pl.ds(0, _BB)], xi_v.at[0], sx).wait()
                fire(kk + 1, 1 - slot, s0)

            return carry

        lax.fori_loop(0, iters, step, 0)

    return k


def kernel(x, tables):
    b, seq, f_dim = x.shape
    xt = x.transpose(1, 2, 0)
    out_t = _sc_embed(b, seq, f_dim)(xt, tables)
    return out_t.transpose(2, 0, 1)


# speedup vs baseline: 1.6338x; 1.6307x over previous
import functools

import jax
import jax.numpy as jnp
from jax import lax
from jax.experimental import pallas as pl
from jax.experimental.pallas import tpu as pltpu
from jax.experimental.pallas import tpu_sc as plsc

_ND = 13
_NF = 26
_EMB = 16
_ROW = _ND + _NF * _EMB

_NC = 2
_NS = 16
_NW = _NC * _NS

_BB = 1024
_NG = _BB // 128
_VP = _BB + 1


def _sc_embed(n_batch, seq, f_dim):
    n_q = n_batch // _BB
    n_emb_units = seq * _NF * n_q
    n_dense_units = seq * n_q
    emb_base, emb_extra = divmod(n_emb_units, _NW)
    dense_base, dense_extra = divmod(n_dense_units, _NW)
    mesh = plsc.VectorSubcoreMesh(core_axis_name="c", subcore_axis_name="s")

    @functools.partial(
        pl.kernel,
        mesh=mesh,
        out_type=jax.ShapeDtypeStruct((seq, _ROW, n_batch), jnp.float32),
        scratch_types=[
            pltpu.VMEM((2, _BB), jnp.float32),
            pltpu.VMEM((2 * _NG, 128), jnp.int32),
            pltpu.VMEM((2 * _BB, _EMB), jnp.float32),
            pltpu.VMEM((2 * _EMB, _VP), jnp.float32),
            pltpu.VMEM((_ND, _BB), jnp.float32),
            pltpu.SemaphoreType.DMA,
            pltpu.SemaphoreType.DMA,
            pltpu.SemaphoreType.DMA,
        ],
        compiler_params=pltpu.CompilerParams(
            use_tc_tiling_on_sc=False, needs_layout_passes=False),
    )
    def k(xt, tbl, out, xi_v, idx_v, emb_v, v_out, dense_v, s0, s1, sx):
        wid = lax.axis_index("s") * _NC + lax.axis_index("c")
        lanes = lax.iota(jnp.int32, 16)
        my_units = emb_base + jnp.where(wid < emb_extra, 1, 0)
        my_dense = dense_base + jnp.where(wid < dense_extra, 1, 0)

        def unit_coords(kk):
            q = jnp.minimum(kk, my_units - 1) * _NW + wid
            s = q // (_NF * n_q)
            r = q % (_NF * n_q)
            f = r // n_q
            b0 = (r % n_q) * _BB
            return s, f, b0

        def fetch_xi(kk, slot):
            s, f, b0 = unit_coords(kk)
            return pltpu.async_copy(
                xt.at[s, _ND + f, pl.ds(b0, _BB)],
                xi_v.at[slot], sx)

        def fire(kk, slot, sem):
            s, f, b0 = unit_coords(kk)
            for j in range(_NG):
                for p in range(8):
                    idx_v[slot * _NG + j, pl.ds(p * 16, 16)] = (
                        lax.convert_element_type(
                            xi_v[slot, pl.ds(j * 128 + p * 16, 16)],
                            jnp.int32))
            return [
                pltpu.async_copy(
                    tbl.at[f].at[idx_v.at[slot * _NG + j]],
                    emb_v.at[pl.ds(slot * _BB + j * 128, 128)], sem)
                for j in range(_NG)
            ]

        def weave_write(kk, slot):
            s, f, b0 = unit_coords(kk)

            def weave(bb, carry2):
                base = jnp.full((16,), bb * 16, dtype=jnp.int32)
                for i in range(16):
                    plsc.store_scatter(
                        v_out, [lanes + slot * _EMB, base + i],
                        emb_v[slot * _BB + bb * 16 + i])
                return carry2

            lax.fori_loop(0, _BB // 16, weave, 0)
            pltpu.sync_copy(
                v_out.at[pl.ds(slot * _EMB, _EMB), pl.ds(0, _BB)],
                out.at[s, pl.ds(_ND + f * _EMB, _EMB), pl.ds(b0, _BB)])

        def dense_loop(kk, carry):
            q = jnp.minimum(kk, my_dense - 1) * _NW + wid
            s = q // n_q
            b0 = (q % n_q) * _BB
            pltpu.sync_copy(xt.at[s, pl.ds(0, _ND), pl.ds(b0, _BB)], dense_v)
            pltpu.sync_copy(
                dense_v, out.at[s, pl.ds(0, _ND), pl.ds(b0, _BB)])
            return carry

        lax.fori_loop(0, dense_base + 1, dense_loop, 0)

        iters = emb_base + 1
        if iters % 2:
            iters += 1

        def drain_xi():
            pltpu.make_async_copy(
                xt.at[0, 0, pl.ds(0, _BB)], xi_v.at[0], sx).wait()

        def drain_gathers(sem):
            for _ in range(_NG):
                pltpu.make_async_copy(
                    tbl.at[0].at[idx_v.at[0]],
                    emb_v.at[pl.ds(0, 128)], sem).wait()

        fetch_xi(0, 0).wait()
        fire(0, 0, s0)
        fetch_xi(1, 1)

        def pair(m, carry):
            for par in (0, 1):
                kk = m * 2 + par
                sem, nsem = (s0, s1) if par == 0 else (s1, s0)

                @pl.when(kk + 1 < iters)
                def _():
                    drain_xi()
                    fire(kk + 1, 1 - par, nsem)

                @pl.when(kk + 2 < iters)
                def _():
                    fetch_xi(kk + 2, par)

                drain_gathers(sem)
                weave_write(kk, par)
            return carry

        lax.fori_loop(0, iters // 2, pair, 0)

    return k


def kernel(x, tables):
    b, seq, f_dim = x.shape
    xt = x.transpose(1, 2, 0)
    out_t = _sc_embed(b, seq, f_dim)(xt, tables)
    return out_t.transpose(2, 0, 1)


# speedup vs baseline: 1.7176x; 1.0512x over previous
import functools

import jax
import jax.numpy as jnp
from jax import lax
from jax.experimental import pallas as pl
from jax.experimental.pallas import tpu as pltpu
from jax.experimental.pallas import tpu_sc as plsc

_ND = 13
_NF = 26
_EMB = 16
_ROW = _ND + _NF * _EMB

_NC = 2
_NS = 16
_NW = _NC * _NS

_BB = 1024
_NG = _BB // 128
_VP = _BB + 1


def _sc_embed(n_batch, seq, f_dim):
    n_q = n_batch // _BB
    n_emb_units = seq * _NF * n_q
    n_dense_units = seq * n_q
    emb_base, emb_extra = divmod(n_emb_units, _NW)
    dense_base, dense_extra = divmod(n_dense_units, _NW)
    mesh = plsc.VectorSubcoreMesh(core_axis_name="c", subcore_axis_name="s")

    @functools.partial(
        pl.kernel,
        mesh=mesh,
        out_type=jax.ShapeDtypeStruct((seq, _ROW, n_batch), jnp.float32),
        scratch_types=[
            pltpu.VMEM((2, _BB), jnp.float32),
            pltpu.VMEM((2 * _NG, 128), jnp.int32),
            pltpu.VMEM((2 * _BB, _EMB), jnp.float32),
            pltpu.VMEM((2 * _EMB, _VP), jnp.float32),
            pltpu.VMEM((_ND, _BB), jnp.float32),
            pltpu.SemaphoreType.DMA,
            pltpu.SemaphoreType.DMA,
            pltpu.SemaphoreType.DMA,
            pltpu.SemaphoreType.DMA,
            pltpu.SemaphoreType.DMA,
        ],
        compiler_params=pltpu.CompilerParams(
            use_tc_tiling_on_sc=False, needs_layout_passes=False),
    )
    def k(xt, tbl, out, xi_v, idx_v, emb_v, v_out, dense_v, s0, s1, sx,
          sw0, sw1):
        wid = lax.axis_index("s") * _NC + lax.axis_index("c")
        lanes = lax.iota(jnp.int32, 16)
        my_units = emb_base + jnp.where(wid < emb_extra, 1, 0)
        my_dense = dense_base + jnp.where(wid < dense_extra, 1, 0)

        def unit_coords(kk):
            q = jnp.minimum(kk, my_units - 1) * _NW + wid
            s = q // (_NF * n_q)
            r = q % (_NF * n_q)
            f = r // n_q
            b0 = (r % n_q) * _BB
            return s, f, b0

        def fetch_xi(kk, slot):
            s, f, b0 = unit_coords(kk)
            return pltpu.async_copy(
                xt.at[s, _ND + f, pl.ds(b0, _BB)],
                xi_v.at[slot], sx)

        def fire(kk, slot, sem):
            s, f, b0 = unit_coords(kk)
            for j in range(_NG):
                for p in range(8):
                    idx_v[slot * _NG + j, pl.ds(p * 16, 16)] = (
                        lax.convert_element_type(
                            xi_v[slot, pl.ds(j * 128 + p * 16, 16)],
                            jnp.int32))
            return [
                pltpu.async_copy(
                    tbl.at[f].at[idx_v.at[slot * _NG + j]],
                    emb_v.at[pl.ds(slot * _BB + j * 128, 128)], sem)
                for j in range(_NG)
            ]

        def drain_write(slot, sem):
            pltpu.make_async_copy(
                v_out.at[pl.ds(slot * _EMB, _EMB), pl.ds(0, _BB)],
                out.at[0, pl.ds(_ND, _EMB), pl.ds(0, _BB)], sem).wait()

        def weave_write(kk, slot, sem):
            s, f, b0 = unit_coords(kk)

            def weave(bb, carry2):
                base = jnp.full((16,), bb * 16, dtype=jnp.int32)
                for i in range(16):
                    plsc.store_scatter(
                        v_out, [lanes + slot * _EMB, base + i],
                        emb_v[slot * _BB + bb * 16 + i])
                return carry2

            lax.fori_loop(0, _BB // 16, weave, 0)
            pltpu.async_copy(
                v_out.at[pl.ds(slot * _EMB, _EMB), pl.ds(0, _BB)],
                out.at[s, pl.ds(_ND + f * _EMB, _EMB), pl.ds(b0, _BB)], sem)

        def dense_loop(kk, carry):
            q = jnp.minimum(kk, my_dense - 1) * _NW + wid
            s = q // n_q
            b0 = (q % n_q) * _BB
            pltpu.sync_copy(xt.at[s, pl.ds(0, _ND), pl.ds(b0, _BB)], dense_v)
            pltpu.sync_copy(
                dense_v, out.at[s, pl.ds(0, _ND), pl.ds(b0, _BB)])
            return carry

        lax.fori_loop(0, dense_base + 1, dense_loop, 0)

        iters = emb_base + 1
        if iters % 2:
            iters += 1

        def drain_xi():
            pltpu.make_async_copy(
                xt.at[0, 0, pl.ds(0, _BB)], xi_v.at[0], sx).wait()

        def drain_gathers(sem):
            for _ in range(_NG):
                pltpu.make_async_copy(
                    tbl.at[0].at[idx_v.at[0]],
                    emb_v.at[pl.ds(0, 128)], sem).wait()

        fetch_xi(0, 0).wait()
        fire(0, 0, s0)
        fetch_xi(1, 1)

        def pair(m, carry):
            for par in (0, 1):
                kk = m * 2 + par
                sem, nsem = (s0, s1) if par == 0 else (s1, s0)
                semw = sw0 if par == 0 else sw1

                @pl.when(kk + 1 < iters)
                def _():
                    drain_xi()
                    fire(kk + 1, 1 - par, nsem)

                @pl.when(kk + 2 < iters)
                def _():
                    fetch_xi(kk + 2, par)

                drain_gathers(sem)

                @pl.when(kk >= 2)
                def _():
                    drain_write(par, semw)

                weave_write(kk, par, semw)
            return carry

        lax.fori_loop(0, iters // 2, pair, 0)
        drain_write(0, sw0)
        drain_write(1, sw1)

    return k


def kernel(x, tables):
    b, seq, f_dim = x.shape
    xt = x.transpose(1, 2, 0)
    out_t = _sc_embed(b, seq, f_dim)(xt, tables)
    return out_t.transpose(2, 0, 1)
